# baseline (speedup 1.0000x reference)
import functools

import jax
import jax.numpy as jnp
from jax.experimental import pallas as pl
from jax.experimental.pallas import tpu as pltpu

IMAGE = 96
NBOX = 71
CK1 = 16
CK2 = 24
KPAD = 72
NCHUNK = KPAD // CK2
COS = (128, 128, 64, 64, 32, 3)


def _decoder_kernel(fidx_ref, zwhat_ref, w1, b1, w2, b2, w3, b3, w4, b4,
                    w5, b5, w6, b6, out_ref, s1, s2, s3, s4, s5):
    f32 = jnp.float32
    nl = zwhat_ref.shape[0]

    fidx = fidx_ref[0]
    ohc = (jax.lax.broadcasted_iota(jnp.int32, (nl, CK1), 0)
           == fidx).astype(f32)
    x = jax.lax.dot_general(ohc, zwhat_ref[...],
                            (((0,), (0,)), ((), ())),
                            preferred_element_type=f32)

    h = 1
    for wref, bref, scr, co in ((w1, b1, s1, 128), (w2, b2, s2, 128),
                                (w3, b3, s3, 64), (w4, b4, s4, 64),
                                (w5, b5, s5, 32)):
        bt = bref[...]
        for a in range(2):
            for bq in range(2):
                y = jax.nn.relu(
                    jnp.dot(x, wref[2 * a + bq],
                            preferred_element_type=f32) + bt)
                scr[:, a::2, bq::2, :] = y.reshape(CK1, h, h, co)
        h *= 2
        x = scr[...].reshape(CK1 * h * h, co)

    y6 = jnp.dot(x, w6[...], preferred_element_type=f32) + b6[...]
    out_ref[0] = y6.reshape(CK1, 32, 32, 12)


def _stn_kernel(idxf_ref, dimg_ref, zwhere_ref, zdep_ref, zpres_ref,
                out_ref):
    c = pl.program_id(1)
    f32 = jnp.float32
    nl = zdep_ref.shape[1]

    idx_full = idxf_ref[0]
    oh_full = (jax.lax.broadcasted_iota(jnp.int32, (nl, KPAD), 0)
               == idx_full).astype(f32)
    dep_g = jax.lax.dot_general(oh_full, zdep_ref[0],
                                (((0,), (0,)), ((), ())),
                                preferred_element_type=f32)
    validv = jax.lax.broadcasted_iota(jnp.int32, (KPAD, 1), 0) < NBOX
    dm = jnp.where(zpres_ref[0] == 1.0, dep_g, f32(-1000.0))
    dm = jnp.where(validv, dm, f32(-1000.0))
    e = jnp.where(validv, jnp.exp(dm - jnp.max(dm)), f32(0.0))
    wts = e / jnp.sum(e)
    rowi = jax.lax.broadcasted_iota(jnp.int32, (CK2, KPAD), 0) + c * CK2
    coli = jax.lax.broadcasted_iota(jnp.int32, (CK2, KPAD), 1)
    sel = (rowi == coli).astype(f32)
    wch = jax.lax.dot_general(sel, wts, (((1,), (0,)), ((), ())),
                              preferred_element_type=f32)

    zwr = zwhere_ref[0]
    eps = f32(1e-9)
    cx = zwr[:, 0:1]
    cy = zwr[:, 1:2]
    sx = 1.0 / (zwr[:, 2:3] + eps)
    sy = 1.0 / (zwr[:, 3:4] + eps)
    tx = (1.0 - 2.0 * cx) * sx
    ty = (1.0 - 2.0 * cy) * sy
    basej = ((jax.lax.broadcasted_iota(jnp.int32, (1, IMAGE), 1)
              .astype(f32) + 0.5) / IMAGE * 2.0 - 1.0)
    ix = ((basej * sx + tx) + 1.0) * 0.5 * 64.0 - 0.5
    iy = ((basej * sy + ty) + 1.0) * 0.5 * 64.0 - 0.5
    yrow = (jax.lax.broadcasted_iota(jnp.int32, (CK2, IMAGE, 64), 2)
            .astype(f32))
    ryw = (jax.nn.relu(1.0 - jnp.abs(iy[:, :, None] - yrow))
           * wch[:, :, None])
    xcol = (jax.lax.broadcasted_iota(jnp.int32, (CK2, 64, IMAGE), 1)
            .astype(f32))
    rxt = jax.nn.relu(1.0 - jnp.abs(ix[:, None, :] - xcol))

    dimg = jax.nn.sigmoid(dimg_ref[0])
    t1 = jax.lax.dot_general(ryw, dimg, (((2,), (1,)), ((0,), (0,))),
                             preferred_element_type=f32)
    accs = []
    for ch in range(3):
        tc = t1[:, :, 64 * ch:64 * (ch + 1)]
        oc = jax.lax.dot_general(tc, rxt, (((2,), (1,)), ((0,), (0,))),
                                 preferred_element_type=f32)
        accs.append(jnp.sum(oc, axis=0)[None])
    acc = jnp.concatenate(accs, axis=0)

    @pl.when(c == 0)
    def _():
        out_ref[0] = acc

    @pl.when(c != 0)
    def _():
        out_ref[0] = out_ref[0] + acc


def kernel(z_what, z_where, z_present, z_depth, indices,
           w1, b1, w2, b2, w3, b3, w4, b4, w5, b5, w6, b6):
    f32 = jnp.float32
    batch = z_what.shape[0]
    nloc = z_what.shape[1]
    nbox = z_where.shape[1]
    pad = KPAD - nbox
    idx = indices.astype(jnp.int32)
    idx_p = jnp.pad(idx, (0, pad))
    idx_full = idx_p.reshape(1, 1, KPAD)

    g = jnp.arange(batch * KPAD, dtype=jnp.int32)
    fidx = (g // KPAD) * nloc + idx_p[g % KPAD]
    nch1 = (batch * KPAD) // CK1
    fidx = fidx.reshape(nch1, 1, CK1)
    zwhat_flat = z_what.reshape(batch * nloc, 64).astype(f32)

    zwhere_p = jnp.pad(z_where.astype(f32), ((0, 0), (0, pad), (0, 0)),
                       constant_values=0.5)
    zpres_p = jnp.pad(z_present.astype(f32), ((0, 0), (0, pad), (0, 0)))

    wqs, bts = [], []
    for w, b in ((w1, b1), (w2, b2), (w3, b3), (w4, b4), (w5, b5)):
        ci, co = w.shape[0], w.shape[1]
        wqs.append(jnp.transpose(w, (2, 3, 0, 1)).reshape(4, ci, co)
                   .astype(f32))
        bts.append(b.reshape(1, co).astype(f32))
    wm6 = jnp.transpose(w6, (0, 2, 3, 1)).reshape(w6.shape[0], 12).astype(f32)
    bt6 = jnp.tile(b6, 4).reshape(1, 12).astype(f32)

    wb_specs = []
    for wq in wqs:
        wb_specs.append(pl.BlockSpec(wq.shape, lambda i: (0, 0, 0)))
        wb_specs.append(pl.BlockSpec((1, wq.shape[2]), lambda i: (0, 0)))
    wb_specs.append(pl.BlockSpec(wm6.shape, lambda i: (0, 0)))
    wb_specs.append(pl.BlockSpec((1, 12), lambda i: (0, 0)))
    wb_inputs = []
    for wq, bt in zip(wqs, bts):
        wb_inputs.extend((wq, bt))
    wb_inputs.extend((wm6, bt6))

    scratch = [pltpu.VMEM((CK1, 2 * hh, 2 * hh, co), f32)
               for hh, co in ((1, 128), (2, 128), (4, 64), (8, 64),
                              (16, 32))]

    dimg = pl.pallas_call(
        _decoder_kernel,
        grid=(nch1,),
        in_specs=[
            pl.BlockSpec((1, 1, CK1), lambda i: (i, 0, 0)),
            pl.BlockSpec((batch * nloc, 64), lambda i: (0, 0)),
        ] + wb_specs,
        out_specs=pl.BlockSpec((1, CK1, 32, 32, 12),
                               lambda i: (i, 0, 0, 0, 0)),
        out_shape=jax.ShapeDtypeStruct((nch1, CK1, 32, 32, 12), f32),
        scratch_shapes=scratch,
    )(fidx, zwhat_flat, *wb_inputs)

    dimg = (dimg.reshape(batch * KPAD, 32, 32, 2, 2, 3)
            .transpose(0, 1, 3, 5, 2, 4)
            .reshape(batch, KPAD, 64, 3 * 64))

    return pl.pallas_call(
        _stn_kernel,
        grid=(batch, NCHUNK),
        in_specs=[
            pl.BlockSpec((1, 1, KPAD), lambda b, c: (0, 0, 0)),
            pl.BlockSpec((1, CK2, 64, 192), lambda b, c: (b, c, 0, 0)),
            pl.BlockSpec((1, CK2, 4), lambda b, c: (b, c, 0)),
            pl.BlockSpec((1, nloc, 1), lambda b, c: (b, 0, 0)),
            pl.BlockSpec((1, KPAD, 1), lambda b, c: (b, 0, 0)),
        ],
        out_specs=pl.BlockSpec((1, 3, IMAGE, IMAGE),
                               lambda b, c: (b, 0, 0, 0)),
        out_shape=jax.ShapeDtypeStruct((batch, 3, IMAGE, IMAGE), f32),
    )(idx_full, dimg, zwhere_p, z_depth.astype(f32), zpres_p)


# speedup vs baseline: 1.1357x; 1.1357x over previous
import functools

import jax
import jax.numpy as jnp
from jax.experimental import pallas as pl
from jax.experimental.pallas import tpu as pltpu

IMAGE = 96
NBOX = 71
CK1 = 32
CK2 = 72
KPAD = 72
NCHUNK = KPAD // CK2
COS = (128, 128, 64, 64, 32, 3)


def _decoder_kernel(fidx_ref, zwhat_ref, w1, b1, w2, b2, w3, b3, w4, b4,
                    w5, b5, w6, b6, out_ref, s1, s2, s3, s4, s5):
    f32 = jnp.float32
    nl = zwhat_ref.shape[0]

    fidx = fidx_ref[0]
    ohc = (jax.lax.broadcasted_iota(jnp.int32, (nl, CK1), 0)
           == fidx).astype(f32)
    x = jax.lax.dot_general(ohc, zwhat_ref[...],
                            (((0,), (0,)), ((), ())),
                            preferred_element_type=f32)

    h = 1
    for wref, bref, scr, co, packed in (
            (w1, b1, s1, 128, True), (w2, b2, s2, 128, True),
            (w3, b3, s3, 64, False), (w4, b4, s4, 64, False),
            (w5, b5, s5, 32, False)):
        bt = bref[...]
        if packed:
            yp = jnp.dot(x, wref[...], preferred_element_type=f32)
            for a in range(2):
                for bq in range(2):
                    q = 2 * a + bq
                    y = jax.nn.relu(yp[:, q * co:(q + 1) * co] + bt)
                    scr[:, a::2, bq::2, :] = y.reshape(CK1, h, h, co)
        else:
            for a in range(2):
                for bq in range(2):
                    y = jax.nn.relu(
                        jnp.dot(x, wref[2 * a + bq],
                                preferred_element_type=f32) + bt)
                    scr[:, a::2, bq::2, :] = y.reshape(CK1, h, h, co)
        h *= 2
        x = scr[...].reshape(CK1 * h * h, co)

    bt6 = b6[...]
    ri = jax.lax.broadcasted_iota(jnp.int32, (32, 64), 0)
    ii = jax.lax.broadcasted_iota(jnp.int32, (32, 64), 1)
    e0 = (ii == 2 * ri).astype(f32)
    e1 = (ii == 2 * ri + 1).astype(f32)
    ecat = jnp.concatenate([e0, e1], axis=0)
    for a in range(2):
        zs = [(jnp.dot(x, w6[2 * a + bq], preferred_element_type=f32)
               + bt6).reshape(CK1 * 32, 32, 3) for bq in range(2)]
        zcat = jnp.concatenate(zs, axis=1)
        va = jax.lax.dot_general(zcat, ecat, (((1,), (0,)), ((), ())),
                                 preferred_element_type=f32)
        out_ref[0, :, a::2, :, :] = va.reshape(CK1, 32, 3, 64)


def _stn_kernel(idxf_ref, dimg_ref, zwhere_ref, zdep_ref, zpres_ref,
                out_ref):
    c = pl.program_id(1)
    f32 = jnp.float32
    nl = zdep_ref.shape[1]

    idx_full = idxf_ref[0]
    oh_full = (jax.lax.broadcasted_iota(jnp.int32, (nl, KPAD), 0)
               == idx_full).astype(f32)
    dep_g = jax.lax.dot_general(oh_full, zdep_ref[0],
                                (((0,), (0,)), ((), ())),
                                preferred_element_type=f32)
    validv = jax.lax.broadcasted_iota(jnp.int32, (KPAD, 1), 0) < NBOX
    dm = jnp.where(zpres_ref[0] == 1.0, dep_g, f32(-1000.0))
    dm = jnp.where(validv, dm, f32(-1000.0))
    e = jnp.where(validv, jnp.exp(dm - jnp.max(dm)), f32(0.0))
    wts = e / jnp.sum(e)
    rowi = jax.lax.broadcasted_iota(jnp.int32, (CK2, KPAD), 0) + c * CK2
    coli = jax.lax.broadcasted_iota(jnp.int32, (CK2, KPAD), 1)
    sel = (rowi == coli).astype(f32)
    wch = jax.lax.dot_general(sel, wts, (((1,), (0,)), ((), ())),
                              preferred_element_type=f32)

    zwr = zwhere_ref[0]
    eps = f32(1e-9)
    cx = zwr[:, 0:1]
    cy = zwr[:, 1:2]
    sx = 1.0 / (zwr[:, 2:3] + eps)
    sy = 1.0 / (zwr[:, 3:4] + eps)
    tx = (1.0 - 2.0 * cx) * sx
    ty = (1.0 - 2.0 * cy) * sy
    basej = ((jax.lax.broadcasted_iota(jnp.int32, (1, IMAGE), 1)
              .astype(f32) + 0.5) / IMAGE * 2.0 - 1.0)
    ix = ((basej * sx + tx) + 1.0) * 0.5 * 64.0 - 0.5
    iy = ((basej * sy + ty) + 1.0) * 0.5 * 64.0 - 0.5
    yrow = (jax.lax.broadcasted_iota(jnp.int32, (CK2, IMAGE, 64), 2)
            .astype(f32))
    ryw = (jax.nn.relu(1.0 - jnp.abs(iy[:, :, None] - yrow))
           * wch[:, :, None])
    xcol = (jax.lax.broadcasted_iota(jnp.int32, (CK2, 64, IMAGE), 1)
            .astype(f32))
    rxt = jax.nn.relu(1.0 - jnp.abs(ix[:, None, :] - xcol))

    dimg = jax.nn.sigmoid(dimg_ref[0])
    t1 = jax.lax.dot_general(ryw, dimg, (((2,), (1,)), ((0,), (0,))),
                             preferred_element_type=f32)
    accs = []
    for ch in range(3):
        tc = t1[:, :, 64 * ch:64 * (ch + 1)]
        oc = jax.lax.dot_general(tc, rxt, (((2,), (1,)), ((0,), (0,))),
                                 preferred_element_type=f32)
        accs.append(jnp.sum(oc, axis=0)[None])
    acc = jnp.concatenate(accs, axis=0)

    @pl.when(c == 0)
    def _():
        out_ref[0] = acc

    @pl.when(c != 0)
    def _():
        out_ref[0] = out_ref[0] + acc


def kernel(z_what, z_where, z_present, z_depth, indices,
           w1, b1, w2, b2, w3, b3, w4, b4, w5, b5, w6, b6):
    f32 = jnp.float32
    batch = z_what.shape[0]
    nloc = z_what.shape[1]
    nbox = z_where.shape[1]
    pad = KPAD - nbox
    idx = indices.astype(jnp.int32)
    idx_p = jnp.pad(idx, (0, pad))
    idx_full = idx_p.reshape(1, 1, KPAD)

    g = jnp.arange(batch * KPAD, dtype=jnp.int32)
    fidx = (g // KPAD) * nloc + idx_p[g % KPAD]
    nch1 = (batch * KPAD) // CK1
    fidx = fidx.reshape(nch1, 1, CK1)
    zwhat_flat = z_what.reshape(batch * nloc, 64).astype(f32)

    zwhere_p = jnp.pad(z_where.astype(f32), ((0, 0), (0, pad), (0, 0)),
                       constant_values=0.5)
    zpres_p = jnp.pad(z_present.astype(f32), ((0, 0), (0, pad), (0, 0)))

    wqs, bts = [], []
    for w, b in ((w1, b1), (w2, b2), (w3, b3), (w4, b4), (w5, b5), (w6, b6)):
        ci, co = w.shape[0], w.shape[1]
        wqs.append(jnp.transpose(w, (2, 3, 0, 1)).reshape(4, ci, co)
                   .astype(f32))
        bts.append(b.reshape(1, co).astype(f32))
    for i in range(2):
        ci, co = wqs[i].shape[1], wqs[i].shape[2]
        wqs[i] = jnp.transpose(wqs[i], (1, 0, 2)).reshape(ci, 4 * co)

    wb_specs = []
    wb_inputs = []
    for wq, bt in zip(wqs, bts):
        wb_specs.append(pl.BlockSpec(wq.shape, lambda i, n=wq.ndim: (0,) * n))
        wb_specs.append(pl.BlockSpec(bt.shape, lambda i: (0, 0)))
        wb_inputs.extend((wq, bt))

    scratch = [pltpu.VMEM((CK1, 2 * hh, 2 * hh, co), f32)
               for hh, co in ((1, 128), (2, 128), (4, 64), (8, 64),
                              (16, 32))]

    dimg = pl.pallas_call(
        _decoder_kernel,
        grid=(nch1,),
        in_specs=[
            pl.BlockSpec((1, 1, CK1), lambda i: (i, 0, 0)),
            pl.BlockSpec((batch * nloc, 64), lambda i: (0, 0)),
        ] + wb_specs,
        out_specs=pl.BlockSpec((1, CK1, 64, 3, 64),
                               lambda i: (i, 0, 0, 0, 0)),
        out_shape=jax.ShapeDtypeStruct((nch1, CK1, 64, 3, 64), f32),
        scratch_shapes=scratch,
    )(fidx, zwhat_flat, *wb_inputs)

    dimg = dimg.reshape(batch, KPAD, 64, 3 * 64)

    return pl.pallas_call(
        _stn_kernel,
        grid=(batch, NCHUNK),
        in_specs=[
            pl.BlockSpec((1, 1, KPAD), lambda b, c: (0, 0, 0)),
            pl.BlockSpec((1, CK2, 64, 192), lambda b, c: (b, c, 0, 0)),
            pl.BlockSpec((1, CK2, 4), lambda b, c: (b, c, 0)),
            pl.BlockSpec((1, nloc, 1), lambda b, c: (b, 0, 0)),
            pl.BlockSpec((1, KPAD, 1), lambda b, c: (b, 0, 0)),
        ],
        out_specs=pl.BlockSpec((1, 3, IMAGE, IMAGE),
                               lambda b, c: (b, 0, 0, 0)),
        out_shape=jax.ShapeDtypeStruct((batch, 3, IMAGE, IMAGE), f32),
    )(idx_full, dimg, zwhere_p, z_depth.astype(f32), zpres_p)
